# baseline (speedup 1.0000x reference)
import jax
import jax.numpy as jnp
from jax.experimental import pallas as pl
from jax.experimental.pallas import tpu as pltpu

_K1 = 928981903
_K2 = 3453687069
_K3 = (_K1 ^ _K2 ^ 0x1BD11BDA) & 0xFFFFFFFF

_ROTS = ((13, 15, 26, 6), (17, 29, 16, 24))
_INJECT = ((1, 2, 1), (2, 0, 2), (0, 1, 3), (1, 2, 4), (2, 0, 5))

_BLOCK_ROWS = 32
_CHUNK = 256
_MERGE = 24


def _rotl(x, r):
    return (x * jnp.uint32(1 << r)) | jax.lax.shift_right_logical(
        x, jnp.uint32(32 - r))


def _threefry_bits(x1):
    ks = (_K1, _K2, _K3)
    x0 = x1 + jnp.uint32(_K1)
    first = True
    for g, (a, b, c) in enumerate(_INJECT):
        for r in _ROTS[g % 2]:
            if first:
                first = False
            else:
                x0 = x0 + x1
            x1 = _rotl(x1, r)
            x1 = x1 ^ x0
        x0 = x0 + jnp.uint32(ks[a])
        x1 = x1 + jnp.uint32((ks[b] + c) & 0xFFFFFFFF)
    return x0 ^ x1


def _gumbel_argmax_kernel(x_ref, o_ref):
    rows, cols = x_ref.shape
    pid = pl.program_id(0)
    n_chunks = cols // _CHUNK
    tail = cols - n_chunks * _CHUNK
    eps = jnp.float32(1e-10)
    big = jnp.int32(2**31 - 1)
    row_iota = jax.lax.broadcasted_iota(jnp.int32, (rows, 1), 0)
    row_term = (row_iota + pid * rows) * cols

    lane = jax.lax.broadcasted_iota(jnp.int32, (rows, _CHUNK), 1)
    k2_s32 = jnp.int32(_K2 - (1 << 32))
    laneK = lane + (row_term + k2_s32)

    def chunk_z(base, width, lane_iota):
        x1 = (lane_iota + base).astype(jnp.uint32)
        bits = _threefry_bits(x1)
        fb = (jax.lax.shift_right_logical(bits, jnp.uint32(9))
              | jnp.uint32(0x3F800000))
        u = jax.lax.bitcast_convert_type(fb, jnp.float32) - jnp.float32(1.0)
        g = -jnp.log(-jnp.log(u + eps) + eps)
        return x_ref[:, pl.ds(base, width)] + g

    def merge2(za, ia, zb, ib):
        bsel = zb > za
        return jnp.where(bsel, zb, za), jnp.where(bsel, ib, ia)

    def merge_tree(pairs):
        while len(pairs) > 1:
            nxt = [merge2(*pairs[k], *pairs[k + 1])
                   for k in range(0, len(pairs) - 1, 2)]
            if len(pairs) % 2:
                nxt.append(pairs[-1])
            pairs = nxt
        return pairs[0]

    n_groups = n_chunks // _MERGE

    def scan_body(i, carry):
        vacc, cacc = carry
        ia = _MERGE * i
        parts = [(chunk_z((ia + k) * _CHUNK, _CHUNK, laneK), ia + k)
                 for k in range(_MERGE)]
        zm, im = merge_tree(parts)
        better = zm > vacc
        return jnp.where(better, zm, vacc), jnp.where(better, im, cacc)

    v0 = jnp.full((rows, _CHUNK), -jnp.inf, dtype=jnp.float32)
    c0 = jnp.zeros((rows, _CHUNK), dtype=jnp.int32)
    vacc, cacc = jax.lax.fori_loop(0, n_groups, scan_body, (v0, c0))
    for ia in range(_MERGE * n_groups, n_chunks):
        za = chunk_z(ia * _CHUNK, _CHUNK, laneK)
        better = za > vacc
        vacc = jnp.where(better, za, vacc)
        cacc = jnp.where(better, ia, cacc)

    colacc = cacc * _CHUNK + lane
    m = jnp.max(vacc, axis=1, keepdims=True)
    if tail:
        lane_t = jax.lax.broadcasted_iota(jnp.int32, (rows, tail), 1)
        zt = chunk_z(n_chunks * _CHUNK, tail, lane_t + (row_term + k2_s32))
        colt = lane_t + n_chunks * _CHUNK
        mt = jnp.max(zt, axis=1, keepdims=True)
        mg = jnp.maximum(m, mt)
        i1 = jnp.min(jnp.where(vacc == mg, colacc, big), axis=1, keepdims=True)
        i2 = jnp.min(jnp.where(zt == mg, colt, big), axis=1, keepdims=True)
        ix = jnp.minimum(i1, i2)
    else:
        ix = jnp.min(jnp.where(vacc == m, colacc, big), axis=1, keepdims=True)

    for c in range(n_chunks):
        o_ref[:, pl.ds(c * _CHUNK, _CHUNK)] = (
            lane == ix - c * _CHUNK).astype(jnp.float32)
    if tail:
        lane_t = jax.lax.broadcasted_iota(jnp.int32, (rows, tail), 1)
        o_ref[:, pl.ds(n_chunks * _CHUNK, tail)] = (
            lane_t == ix - n_chunks * _CHUNK).astype(jnp.float32)


@jax.jit
def kernel(logits):
    n_rows, n_cols = logits.shape
    return pl.pallas_call(
        _gumbel_argmax_kernel,
        out_shape=jax.ShapeDtypeStruct((n_rows, n_cols), jnp.float32),
        grid=(n_rows // _BLOCK_ROWS,),
        in_specs=[pl.BlockSpec((_BLOCK_ROWS, n_cols), lambda i: (i, 0))],
        out_specs=pl.BlockSpec((_BLOCK_ROWS, n_cols), lambda i: (i, 0)),
        compiler_params=pltpu.CompilerParams(
            dimension_semantics=("parallel",)),
    )(logits)


# speedup vs baseline: 1.0111x; 1.0111x over previous
import jax
import jax.numpy as jnp
from jax.experimental import pallas as pl
from jax.experimental.pallas import tpu as pltpu

_K1 = 928981903
_K2 = 3453687069
_K3 = (_K1 ^ _K2 ^ 0x1BD11BDA) & 0xFFFFFFFF

_ROTS = ((13, 15, 26, 6), (17, 29, 16, 24))
_INJECT = ((1, 2, 1), (2, 0, 2), (0, 1, 3), (1, 2, 4), (2, 0, 5))

_BLOCK_ROWS = 16
_CHUNK = 512
_MERGE = 24


def _rotl(x, r):
    return (x * jnp.uint32(1 << r)) | jax.lax.shift_right_logical(
        x, jnp.uint32(32 - r))


def _threefry_bits(x1):
    ks = (_K1, _K2, _K3)
    x0 = x1 + jnp.uint32(_K1)
    first = True
    for g, (a, b, c) in enumerate(_INJECT):
        for r in _ROTS[g % 2]:
            if first:
                first = False
            else:
                x0 = x0 + x1
            x1 = _rotl(x1, r)
            x1 = x1 ^ x0
        x0 = x0 + jnp.uint32(ks[a])
        x1 = x1 + jnp.uint32((ks[b] + c) & 0xFFFFFFFF)
    return x0 ^ x1


def _gumbel_argmax_kernel(x_ref, o_ref):
    rows, cols = x_ref.shape
    pid = pl.program_id(0)
    n_chunks = cols // _CHUNK
    tail = cols - n_chunks * _CHUNK
    eps = jnp.float32(1e-10)
    big = jnp.int32(2**31 - 1)
    row_iota = jax.lax.broadcasted_iota(jnp.int32, (rows, 1), 0)
    row_term = (row_iota + pid * rows) * cols

    lane = jax.lax.broadcasted_iota(jnp.int32, (rows, _CHUNK), 1)
    k2_s32 = jnp.int32(_K2 - (1 << 32))
    laneK = lane + (row_term + k2_s32)

    def chunk_z(base, width, lane_iota):
        x1 = (lane_iota + base).astype(jnp.uint32)
        bits = _threefry_bits(x1)
        fb = (jax.lax.shift_right_logical(bits, jnp.uint32(9))
              | jnp.uint32(0x3F800000))
        u = jax.lax.bitcast_convert_type(fb, jnp.float32) - jnp.float32(1.0)
        g = -jnp.log(-jnp.log(u + eps) + eps)
        return x_ref[:, pl.ds(base, width)] + g

    def merge2(za, ia, zb, ib):
        bsel = zb > za
        return jnp.where(bsel, zb, za), jnp.where(bsel, ib, ia)

    def merge_tree(pairs):
        while len(pairs) > 1:
            nxt = [merge2(*pairs[k], *pairs[k + 1])
                   for k in range(0, len(pairs) - 1, 2)]
            if len(pairs) % 2:
                nxt.append(pairs[-1])
            pairs = nxt
        return pairs[0]

    n_groups = n_chunks // _MERGE

    def scan_body(i, carry):
        vacc, cacc = carry
        ia = _MERGE * i
        parts = [(chunk_z((ia + k) * _CHUNK, _CHUNK, laneK), ia + k)
                 for k in range(_MERGE)]
        zm, im = merge_tree(parts)
        better = zm > vacc
        return jnp.where(better, zm, vacc), jnp.where(better, im, cacc)

    v0 = jnp.full((rows, _CHUNK), -jnp.inf, dtype=jnp.float32)
    c0 = jnp.zeros((rows, _CHUNK), dtype=jnp.int32)
    vacc, cacc = jax.lax.fori_loop(0, n_groups, scan_body, (v0, c0))
    for ia in range(_MERGE * n_groups, n_chunks):
        za = chunk_z(ia * _CHUNK, _CHUNK, laneK)
        better = za > vacc
        vacc = jnp.where(better, za, vacc)
        cacc = jnp.where(better, ia, cacc)

    colacc = cacc * _CHUNK + lane
    m = jnp.max(vacc, axis=1, keepdims=True)
    if tail:
        lane_t = jax.lax.broadcasted_iota(jnp.int32, (rows, tail), 1)
        zt = chunk_z(n_chunks * _CHUNK, tail, lane_t + (row_term + k2_s32))
        colt = lane_t + n_chunks * _CHUNK
        mt = jnp.max(zt, axis=1, keepdims=True)
        mg = jnp.maximum(m, mt)
        i1 = jnp.min(jnp.where(vacc == mg, colacc, big), axis=1, keepdims=True)
        i2 = jnp.min(jnp.where(zt == mg, colt, big), axis=1, keepdims=True)
        ix = jnp.minimum(i1, i2)
    else:
        ix = jnp.min(jnp.where(vacc == m, colacc, big), axis=1, keepdims=True)

    for c in range(n_chunks):
        o_ref[:, pl.ds(c * _CHUNK, _CHUNK)] = (
            lane == ix - c * _CHUNK).astype(jnp.float32)
    if tail:
        lane_t = jax.lax.broadcasted_iota(jnp.int32, (rows, tail), 1)
        o_ref[:, pl.ds(n_chunks * _CHUNK, tail)] = (
            lane_t == ix - n_chunks * _CHUNK).astype(jnp.float32)


@jax.jit
def kernel(logits):
    n_rows, n_cols = logits.shape
    return pl.pallas_call(
        _gumbel_argmax_kernel,
        out_shape=jax.ShapeDtypeStruct((n_rows, n_cols), jnp.float32),
        grid=(n_rows // _BLOCK_ROWS,),
        in_specs=[pl.BlockSpec((_BLOCK_ROWS, n_cols), lambda i: (i, 0))],
        out_specs=pl.BlockSpec((_BLOCK_ROWS, n_cols), lambda i: (i, 0)),
        compiler_params=pltpu.CompilerParams(
            dimension_semantics=("parallel",)),
    )(logits)
